# speedup vs baseline: 4.9579x; 4.9579x over previous
import functools

import jax
import jax.numpy as jnp
from jax import lax
from jax.experimental import pallas as pl
from jax.experimental.pallas import tpu as pltpu
from jax.experimental.pallas import tpu_sc as plsc

_VOCAB = 100000
_EMBED = 128
_TRAIN = 10000
_FIXED = _VOCAB - _TRAIN

_LANES = 16
_NC = 2
_NS = 16
_NW = _NC * _NS

_K = 128

_CROWS = 2000


def _concat_tables(fixed3, train3):
    nf = fixed3.shape[0]
    nt = train3.shape[0]
    grid = nf + nt
    rows = fixed3.shape[1]

    def body(f_ref, t_ref, o_ref):
        i = pl.program_id(0)

        @pl.when(i < nf)
        def _():
            o_ref[...] = f_ref[0]

        @pl.when(i >= nf)
        def _():
            o_ref[...] = t_ref[0]

    return pl.pallas_call(
        body,
        grid=(grid,),
        in_specs=[
            pl.BlockSpec((1, rows, _EMBED), lambda i: (jnp.minimum(i, nf - 1), 0, 0)),
            pl.BlockSpec((1, rows, _EMBED), lambda i: (jnp.maximum(i - nf, 0), 0, 0)),
        ],
        out_specs=pl.BlockSpec((rows, _EMBED), lambda i: (i, 0)),
        out_shape=jax.ShapeDtypeStruct((grid * rows, _EMBED), jnp.float32),
    )(fixed3, train3)


def _sc_gather(table, ids_flat):
    batch = ids_flat.shape[0]
    b_per_w = batch // _NW
    n_chunks = b_per_w // _K
    mesh = plsc.VectorSubcoreMesh(core_axis_name="c", subcore_axis_name="s")

    @functools.partial(
        pl.kernel,
        mesh=mesh,
        out_type=jax.ShapeDtypeStruct((batch, _EMBED), jnp.float32),
        scratch_types=[
            pltpu.VMEM((_K,), jnp.int32),
            pltpu.VMEM((_K, _EMBED), jnp.float32),
            pltpu.SemaphoreType.DMA,
        ],
    )
    def k(table_hbm, ids_hbm, out_hbm, idx_v, rows_v, sem):
        wid = lax.axis_index("s") * _NC + lax.axis_index("c")
        base = wid * b_per_w

        def chunk(g, carry):
            start = base + g * _K
            pltpu.sync_copy(ids_hbm.at[pl.ds(start, _K)], idx_v)
            for j in range(_K // _LANES):
                sl = pl.ds(j * _LANES, _LANES)
                idx_v[sl] = jnp.clip(idx_v[sl], 0, _VOCAB - 1)
            pltpu.async_copy(table_hbm.at[idx_v], rows_v, sem).wait()
            pltpu.sync_copy(rows_v, out_hbm.at[pl.ds(start, _K)])
            return carry

        lax.fori_loop(0, n_chunks, chunk, 0)

    return k(table, ids_flat)


def kernel(input_ids, fixed_embedding, train_embedding):
    s0, s1 = input_ids.shape
    fixed3 = fixed_embedding.reshape(_FIXED // _CROWS, _CROWS, _EMBED)
    train3 = train_embedding.reshape(_TRAIN // _CROWS, _CROWS, _EMBED)
    table = _concat_tables(fixed3, train3)
    out = _sc_gather(table, input_ids.reshape(-1))
    return out.reshape(s0, s1, _EMBED)


# speedup vs baseline: 5.7824x; 1.1663x over previous
import functools

import jax
import jax.numpy as jnp
from jax import lax
from jax.experimental import pallas as pl
from jax.experimental.pallas import tpu as pltpu
from jax.experimental.pallas import tpu_sc as plsc

_VOCAB = 100000
_EMBED = 128
_TRAIN = 10000
_FIXED = _VOCAB - _TRAIN

_LANES = 16
_NC = 2
_NS = 16
_NW = _NC * _NS

_K = 128

_CROWS = 2000


def _concat_tables(fixed3, train3):
    nf = fixed3.shape[0]
    nt = train3.shape[0]
    grid = nf + nt
    rows = fixed3.shape[1]

    def body(f_ref, t_ref, o_ref):
        i = pl.program_id(0)

        @pl.when(i < nf)
        def _():
            o_ref[...] = f_ref[0]

        @pl.when(i >= nf)
        def _():
            o_ref[...] = t_ref[0]

    return pl.pallas_call(
        body,
        grid=(grid,),
        in_specs=[
            pl.BlockSpec((1, rows, _EMBED), lambda i: (jnp.minimum(i, nf - 1), 0, 0)),
            pl.BlockSpec((1, rows, _EMBED), lambda i: (jnp.maximum(i - nf, 0), 0, 0)),
        ],
        out_specs=pl.BlockSpec((rows, _EMBED), lambda i: (i, 0)),
        out_shape=jax.ShapeDtypeStruct((grid * rows, _EMBED), jnp.float32),
    )(fixed3, train3)


_NBUF = 5
_DEPTH = 3


def _sc_gather(table, ids2):
    b_per_w = ids2.shape[1]
    batch = ids2.shape[0] * b_per_w
    n_chunks = b_per_w // _K
    mesh = plsc.VectorSubcoreMesh(core_axis_name="c", subcore_axis_name="s")

    @functools.partial(
        pl.kernel,
        mesh=mesh,
        out_type=jax.ShapeDtypeStruct((batch, _EMBED), jnp.float32),
        scratch_types=[pltpu.VMEM((b_per_w,), jnp.int32)]
        + [pltpu.VMEM((_K, _EMBED), jnp.float32) for _ in range(_NBUF)]
        + [pltpu.SemaphoreType.DMA for _ in range(_NBUF)],
    )
    def k(table_hbm, ids_hbm, out_hbm, idx_all, *rest):
        bufs = rest[:_NBUF]
        sems = rest[_NBUF:]
        wid = lax.axis_index("s") * _NC + lax.axis_index("c")
        base = wid * b_per_w

        pltpu.sync_copy(ids_hbm.at[wid], idx_all)

        def clip_chunk(g, carry):
            sl = pl.ds(g * _LANES, _LANES)
            idx_all[sl] = jnp.clip(idx_all[sl], 0, _VOCAB - 1)
            return carry

        lax.fori_loop(0, b_per_w // _LANES, clip_chunk, 0)

        def fire(c, buf, sem):
            pltpu.async_copy(
                table_hbm.at[idx_all.at[pl.ds(c * _K, _K)]], buf, sem
            )

        def drain(c, buf, sem):
            pltpu.make_async_copy(
                table_hbm.at[idx_all.at[pl.ds(c * _K, _K)]], buf, sem
            ).wait()

        for t in range(_DEPTH):
            fire(t, bufs[t], sems[t])

        def group(m, carry):
            for b in range(_NBUF):
                c = m * _NBUF + b
                cn = jnp.minimum(c + _DEPTH, n_chunks - 1)
                bn = (b + _DEPTH) % _NBUF
                drain(c, bufs[b], sems[b])
                fire(cn, bufs[bn], sems[bn])
                pltpu.sync_copy(bufs[b], out_hbm.at[pl.ds(base + c * _K, _K)])
            return carry

        lax.fori_loop(0, n_chunks // _NBUF, group, 0)

        for t in range(_DEPTH):
            b = (n_chunks + t) % _NBUF
            drain(n_chunks - 1, bufs[b], sems[b])

    return k(table, ids2)


def kernel(input_ids, fixed_embedding, train_embedding):
    s0, s1 = input_ids.shape
    fixed3 = fixed_embedding.reshape(_FIXED // _CROWS, _CROWS, _EMBED)
    train3 = train_embedding.reshape(_TRAIN // _CROWS, _CROWS, _EMBED)
    table = _concat_tables(fixed3, train3)
    ids2 = input_ids.reshape(_NW, (s0 * s1) // _NW)
    out = _sc_gather(table, ids2)
    return out.reshape(s0, s1, _EMBED)
